# baseline (speedup 1.0000x reference)
import functools

import jax
import jax.numpy as jnp
from jax import lax
from jax.experimental import pallas as pl
from jax.experimental.pallas import tpu as pltpu
from jax.experimental.pallas import tpu_sc as plsc

N = 10000
E = 320000
D = 128
H = 16
C = 40

NC, NS, L = 2, 16, 16
NW = NC * NS
N_PAD = 10240
KC = 512
NCH = E // KC
CH_BASE = NCH // NW
CH_EXTRA = NCH - CH_BASE * NW
RPT = N_PAD // NS
NSLOT = 4
NSTEP = 4 * ((CH_BASE + 1 + 2) // 4 + 1)


def _worker_range(wid):
    nch = CH_BASE + (wid < CH_EXTRA).astype(jnp.int32)
    ch0 = wid * CH_BASE + jnp.minimum(wid, CH_EXTRA)
    return ch0, nch


def _load_idx(e2d, sbuf, dbuf, sem_i, ch0, wid):
    pltpu.async_copy(e2d.at[0, pl.ds(ch0 * KC, CH_BASE * KC)],
                     sbuf.at[pl.ds(0, CH_BASE * KC)], sem_i)
    pltpu.async_copy(e2d.at[1, pl.ds(ch0 * KC, CH_BASE * KC)],
                     dbuf.at[pl.ds(0, CH_BASE * KC)], sem_i)

    @pl.when(wid < CH_EXTRA)
    def _():
        pltpu.async_copy(e2d.at[0, pl.ds((ch0 + CH_BASE) * KC, KC)],
                         sbuf.at[pl.ds(CH_BASE * KC, KC)], sem_i)
        pltpu.async_copy(e2d.at[1, pl.ds((ch0 + CH_BASE) * KC, KC)],
                         dbuf.at[pl.ds(CH_BASE * KC, KC)], sem_i)


def _drain_idx(e2d, sbuf, dbuf, sem_i, ch0, wid):
    pltpu.make_async_copy(e2d.at[0, pl.ds(ch0 * KC, CH_BASE * KC)],
                          sbuf.at[pl.ds(0, CH_BASE * KC)], sem_i).wait()
    pltpu.make_async_copy(e2d.at[1, pl.ds(ch0 * KC, CH_BASE * KC)],
                          dbuf.at[pl.ds(0, CH_BASE * KC)], sem_i).wait()

    @pl.when(wid < CH_EXTRA)
    def _():
        pltpu.make_async_copy(e2d.at[0, pl.ds((ch0 + CH_BASE) * KC, KC)],
                              sbuf.at[pl.ds(CH_BASE * KC, KC)], sem_i).wait()
        pltpu.make_async_copy(e2d.at[1, pl.ds((ch0 + CH_BASE) * KC, KC)],
                              dbuf.at[pl.ds(CH_BASE * KC, KC)], sem_i).wait()


def _agg_pipeline(table, sbuf, dbuf, rows, nch, accum, sem_g, sem_s):
    def step(c, q):
        q2 = (q + 2) % NSLOT

        @pl.when(jnp.logical_and(c >= 2, c - 2 < nch))
        def _():
            pltpu.make_async_copy(
                rows.at[q2], accum.at[dbuf.at[pl.ds(0, KC)]],
                sem_s[q2]).wait()

        @pl.when(c + 2 < nch)
        def _():
            pltpu.async_copy(table.at[sbuf.at[pl.ds((c + 2) * KC, KC)]],
                             rows.at[q2], sem_g[q2])

        @pl.when(c < nch)
        def _():
            pltpu.make_async_copy(
                table.at[sbuf.at[pl.ds(0, KC)]], rows.at[q],
                sem_g[q]).wait()
            pltpu.async_copy(rows.at[q],
                             accum.at[dbuf.at[pl.ds(c * KC, KC)]],
                             sem_s[q], add=True)

    pltpu.async_copy(table.at[sbuf.at[pl.ds(0, KC)]], rows.at[0], sem_g[0])
    pltpu.async_copy(table.at[sbuf.at[pl.ds(KC, KC)]], rows.at[1], sem_g[1])

    def outer(i, _):
        for q in range(NSLOT):
            step(i * NSLOT + q, q)
        return 0
    lax.fori_loop(0, NSTEP // NSLOT, outer, 0)


def _zero_fill(buf, n):
    def f(i, _):
        buf[i] = jnp.zeros((H,), jnp.float32)
        return 0
    lax.fori_loop(0, n, f, 0)


def _sc_l1_body(table, e2d, agg_out, deg_out,
                sbuf, dbuf, rows, zbuf, deg_local, dred, accum, deg_stage,
                sem_i, sem_p, sg0, sg1, sg2, sg3, ss0, ss1, ss2, ss3):
    sem_g, sem_s = (sg0, sg1, sg2, sg3), (ss0, ss1, ss2, ss3)
    cid = lax.axis_index("c")
    sid = lax.axis_index("s")
    wid = sid * NC + cid
    row0 = sid * RPT
    ch0, nch = _worker_range(wid)

    _load_idx(e2d, sbuf, dbuf, sem_i, ch0, wid)
    _zero_fill(zbuf, RPT)
    pltpu.sync_copy(zbuf, accum.at[pl.ds(row0, RPT)])

    def fz(i, _):
        deg_local[pl.ds(i * H, H)] = jnp.zeros((H,), jnp.float32)
        return 0
    lax.fori_loop(0, N_PAD // H, fz, 0)
    _drain_idx(e2d, sbuf, dbuf, sem_i, ch0, wid)
    plsc.subcore_barrier()

    _agg_pipeline(table, sbuf, dbuf, rows, nch, accum, sem_g, sem_s)

    ones16 = jnp.ones((H,), jnp.float32)

    def hist(j, _):
        dvec = dbuf[pl.ds(j * H, H)]
        plsc.addupdate_scatter(deg_local, [dvec], ones16)
        return 0
    lax.fori_loop(0, nch * (KC // H), hist, 0)

    pltpu.sync_copy(deg_local, deg_stage.at[sid])
    plsc.subcore_barrier()
    for k in range(NS):
        pltpu.async_copy(deg_stage.at[k, pl.ds(row0, RPT)], dred.at[k],
                         sem_p)
    for k in range(NS):
        pltpu.make_async_copy(deg_stage.at[k, pl.ds(row0, RPT)],
                              dred.at[k], sem_p).wait()

    def fr(i, _):
        acc = dred[0, pl.ds(i * H, H)]
        for k in range(1, NS):
            acc = acc + dred[k, pl.ds(i * H, H)]
        deg_local[pl.ds(i * H, H)] = acc
        return 0
    lax.fori_loop(0, RPT // H, fr, 0)

    out_off = cid * N_PAD + row0
    pltpu.sync_copy(deg_local.at[pl.ds(0, RPT)],
                    deg_out.at[pl.ds(out_off, RPT)])
    pltpu.sync_copy(accum.at[pl.ds(row0, RPT)], zbuf)
    pltpu.sync_copy(zbuf, agg_out.at[pl.ds(out_off, RPT)])


def _sc_l2_body(agg_in, deg_in, e2d, b1h, mu_out,
                sbuf, dbuf, rows, zbuf, a0, a1, d0, d1, b1v, htab, accum,
                sem_i, sem_p, sg0, sg1, sg2, sg3, ss0, ss1, ss2, ss3):
    sem_g, sem_s = (sg0, sg1, sg2, sg3), (ss0, ss1, ss2, ss3)
    cid = lax.axis_index("c")
    sid = lax.axis_index("s")
    wid = sid * NC + cid
    row0 = sid * RPT
    ch0, nch = _worker_range(wid)

    _load_idx(e2d, sbuf, dbuf, sem_i, ch0, wid)
    pltpu.async_copy(agg_in.at[pl.ds(row0, RPT)], a0, sem_p)
    pltpu.async_copy(agg_in.at[pl.ds(N_PAD + row0, RPT)], a1, sem_p)
    pltpu.async_copy(deg_in.at[pl.ds(row0, RPT)], d0, sem_p)
    pltpu.async_copy(deg_in.at[pl.ds(N_PAD + row0, RPT)], d1, sem_p)
    pltpu.async_copy(b1h, b1v, sem_p)

    _zero_fill(zbuf, RPT)
    pltpu.sync_copy(zbuf, accum.at[pl.ds(row0, RPT)])

    pltpu.make_async_copy(agg_in.at[pl.ds(row0, RPT)], a0, sem_p).wait()
    pltpu.make_async_copy(agg_in.at[pl.ds(row0, RPT)], a1, sem_p).wait()
    pltpu.make_async_copy(deg_in.at[pl.ds(row0, RPT)], d0, sem_p).wait()
    pltpu.make_async_copy(deg_in.at[pl.ds(row0, RPT)], d1, sem_p).wait()
    pltpu.make_async_copy(b1h, b1v, sem_p).wait()

    bvec = b1v[0]

    def hblk(i, _):
        dv = jnp.maximum(d0[pl.ds(i * H, H)] + d1[pl.ds(i * H, H)], 1.0)
        rv = jnp.ones((H,), jnp.float32) / dv
        for m in range(H):
            r = i * H + m
            rm = jnp.full((H,), rv[m], jnp.float32)
            a0[r] = jnp.maximum((a0[r] + a1[r]) * rm + bvec, 0.0)
        return 0
    lax.fori_loop(0, RPT // H, hblk, 0)
    pltpu.sync_copy(a0, htab.at[pl.ds(row0, RPT)])
    _drain_idx(e2d, sbuf, dbuf, sem_i, ch0, wid)
    plsc.subcore_barrier()

    _agg_pipeline(htab, sbuf, dbuf, rows, nch, accum, sem_g, sem_s)

    plsc.subcore_barrier()
    pltpu.sync_copy(accum.at[pl.ds(row0, RPT)], zbuf)

    def mblk(i, _):
        dv = jnp.maximum(d0[pl.ds(i * H, H)] + d1[pl.ds(i * H, H)], 1.0)
        rv = jnp.ones((H,), jnp.float32) / dv
        for m in range(H):
            r = i * H + m
            zbuf[r] = zbuf[r] * jnp.full((H,), rv[m], jnp.float32)
        return 0
    lax.fori_loop(0, RPT // H, mblk, 0)
    pltpu.sync_copy(zbuf, mu_out.at[pl.ds(cid * N_PAD + row0, RPT)])


_SC_MESH = plsc.VectorSubcoreMesh(
    core_axis_name="c", subcore_axis_name="s",
    num_cores=NC, num_subcores=NS)
_SC_PARAMS = pltpu.CompilerParams(use_tc_tiling_on_sc=False,
                                  needs_layout_passes=False)


def _make_sc_l1():
    scratch = [
        pltpu.VMEM(((CH_BASE + 1) * KC,), jnp.int32),
        pltpu.VMEM(((CH_BASE + 1) * KC,), jnp.int32),
        pltpu.VMEM((NSLOT, KC, H), jnp.float32),
        pltpu.VMEM((RPT, H), jnp.float32),
        pltpu.VMEM((N_PAD,), jnp.float32),
        pltpu.VMEM((NS, RPT), jnp.float32),
        pltpu.VMEM_SHARED((N_PAD, H), jnp.float32),
        pltpu.VMEM_SHARED((NS, N_PAD), jnp.float32),
    ] + [pltpu.SemaphoreType.DMA] * 10
    return pl.kernel(
        _sc_l1_body,
        out_type=(jax.ShapeDtypeStruct((NC * N_PAD, H), jnp.float32),
                  jax.ShapeDtypeStruct((NC * N_PAD,), jnp.float32)),
        mesh=_SC_MESH,
        scratch_types=scratch,
        compiler_params=_SC_PARAMS,
    )


def _make_sc_l2():
    scratch = [
        pltpu.VMEM(((CH_BASE + 1) * KC,), jnp.int32),
        pltpu.VMEM(((CH_BASE + 1) * KC,), jnp.int32),
        pltpu.VMEM((NSLOT, KC, H), jnp.float32),
        pltpu.VMEM((RPT, H), jnp.float32),
        pltpu.VMEM((RPT, H), jnp.float32),
        pltpu.VMEM((RPT, H), jnp.float32),
        pltpu.VMEM((RPT,), jnp.float32),
        pltpu.VMEM((RPT,), jnp.float32),
        pltpu.VMEM((1, H), jnp.float32),
        pltpu.VMEM_SHARED((N_PAD, H), jnp.float32),
        pltpu.VMEM_SHARED((N_PAD, H), jnp.float32),
    ] + [pltpu.SemaphoreType.DMA] * 10
    return pl.kernel(
        _sc_l2_body,
        out_type=jax.ShapeDtypeStruct((NC * N_PAD, H), jnp.float32),
        mesh=_SC_MESH,
        scratch_types=scratch,
        compiler_params=_SC_PARAMS,
    )


def _mm_body(x_ref, w_ref, o_ref):
    o_ref[...] = jnp.dot(x_ref[...], w_ref[...],
                         preferred_element_type=jnp.float32)


def _out_body(m0, m1, w_ref, b_ref, o_ref):
    z = jnp.dot(m0[...] + m1[...], w_ref[...],
                preferred_element_type=jnp.float32) + b_ref[...]
    m = jnp.max(z, axis=1, keepdims=True)
    lse = jnp.log(jnp.sum(jnp.exp(z - m), axis=1, keepdims=True)) + m
    o_ref[...] = z - lse


def kernel(x, edge_index, W1, b1, W2, b2):
    e2d = edge_index
    NB = N_PAD // 1024

    y1p = pl.pallas_call(
        _mm_body,
        grid=(5,),
        in_specs=[pl.BlockSpec((2048, D), lambda i: (i, 0)),
                  pl.BlockSpec((D, H), lambda i: (0, 0))],
        out_specs=pl.BlockSpec((2048, H), lambda i: (i, 0)),
        out_shape=jax.ShapeDtypeStruct((N_PAD, H), jnp.float32),
    )(x, W1)

    agg1, degp = _make_sc_l1()(y1p, e2d)

    mu2 = _make_sc_l2()(agg1, degp, e2d, b1.reshape(1, H))

    bspec = lambda off: pl.BlockSpec((2048, H), lambda i: (i + off, 0))
    out = pl.pallas_call(
        _out_body,
        grid=(5,),
        in_specs=[bspec(0), bspec(5),
                  pl.BlockSpec((H, C), lambda i: (0, 0)),
                  pl.BlockSpec((1, C), lambda i: (0, 0))],
        out_specs=pl.BlockSpec((2048, C), lambda i: (i, 0)),
        out_shape=jax.ShapeDtypeStruct((N, C), jnp.float32),
    )(mu2, mu2, W2, b2.reshape(1, C))
    return out


# speedup vs baseline: 1.0337x; 1.0337x over previous
import functools

import jax
import jax.numpy as jnp
from jax import lax
from jax.experimental import pallas as pl
from jax.experimental.pallas import tpu as pltpu
from jax.experimental.pallas import tpu_sc as plsc

N = 10000
E = 320000
D = 128
H = 16
C = 40

NC, NS, L = 2, 16, 16
NW = NC * NS
N_PAD = 10240
KC = 512
NCH = E // KC
CH_BASE = NCH // NW
CH_EXTRA = NCH - CH_BASE * NW
RPT = N_PAD // NS
NSLOT = 4
NSTEP = 4 * ((CH_BASE + 1 + 2) // 4 + 1)


def _worker_range(wid):
    nch = CH_BASE + (wid < CH_EXTRA).astype(jnp.int32)
    ch0 = wid * CH_BASE + jnp.minimum(wid, CH_EXTRA)
    return ch0, nch


def _load_idx(e2d, sbuf, dbuf, sem_i, ch0, wid):
    pltpu.async_copy(e2d.at[0, pl.ds(ch0 * KC, CH_BASE * KC)],
                     sbuf.at[pl.ds(0, CH_BASE * KC)], sem_i)
    pltpu.async_copy(e2d.at[1, pl.ds(ch0 * KC, CH_BASE * KC)],
                     dbuf.at[pl.ds(0, CH_BASE * KC)], sem_i)

    @pl.when(wid < CH_EXTRA)
    def _():
        pltpu.async_copy(e2d.at[0, pl.ds((ch0 + CH_BASE) * KC, KC)],
                         sbuf.at[pl.ds(CH_BASE * KC, KC)], sem_i)
        pltpu.async_copy(e2d.at[1, pl.ds((ch0 + CH_BASE) * KC, KC)],
                         dbuf.at[pl.ds(CH_BASE * KC, KC)], sem_i)


def _drain_idx(e2d, sbuf, dbuf, sem_i, ch0, wid):
    pltpu.make_async_copy(e2d.at[0, pl.ds(ch0 * KC, CH_BASE * KC)],
                          sbuf.at[pl.ds(0, CH_BASE * KC)], sem_i).wait()
    pltpu.make_async_copy(e2d.at[1, pl.ds(ch0 * KC, CH_BASE * KC)],
                          dbuf.at[pl.ds(0, CH_BASE * KC)], sem_i).wait()

    @pl.when(wid < CH_EXTRA)
    def _():
        pltpu.make_async_copy(e2d.at[0, pl.ds((ch0 + CH_BASE) * KC, KC)],
                              sbuf.at[pl.ds(CH_BASE * KC, KC)], sem_i).wait()
        pltpu.make_async_copy(e2d.at[1, pl.ds((ch0 + CH_BASE) * KC, KC)],
                              dbuf.at[pl.ds(CH_BASE * KC, KC)], sem_i).wait()


def _agg_pipeline(table, sbuf, dbuf, rows, nch, accum, deg_local,
                  sem_g, sem_s):
    ones16 = jnp.ones((H,), jnp.float32)

    def step(c, q):
        q2 = (q + 2) % NSLOT

        @pl.when(jnp.logical_and(c >= 2, c - 2 < nch))
        def _():
            pltpu.make_async_copy(
                rows.at[q2], accum.at[dbuf.at[pl.ds(0, KC)]],
                sem_s[q2]).wait()

        @pl.when(c + 2 < nch)
        def _():
            pltpu.async_copy(table.at[sbuf.at[pl.ds((c + 2) * KC, KC)]],
                             rows.at[q2], sem_g[q2])

        @pl.when(c < nch)
        def _():
            if deg_local is not None:
                for k in range(KC // H):
                    dvec = dbuf[pl.ds(c * KC + k * H, H)]
                    plsc.addupdate_scatter(deg_local, [dvec], ones16)
            pltpu.make_async_copy(
                table.at[sbuf.at[pl.ds(0, KC)]], rows.at[q],
                sem_g[q]).wait()
            pltpu.async_copy(rows.at[q],
                             accum.at[dbuf.at[pl.ds(c * KC, KC)]],
                             sem_s[q], add=True)

    pltpu.async_copy(table.at[sbuf.at[pl.ds(0, KC)]], rows.at[0], sem_g[0])
    pltpu.async_copy(table.at[sbuf.at[pl.ds(KC, KC)]], rows.at[1], sem_g[1])

    def outer(i, _):
        for q in range(NSLOT):
            step(i * NSLOT + q, q)
        return 0
    lax.fori_loop(0, NSTEP // NSLOT, outer, 0)


def _zero_fill(buf, n):
    def f(i, _):
        buf[i] = jnp.zeros((H,), jnp.float32)
        return 0
    lax.fori_loop(0, n, f, 0)


def _sc_l1_body(table, e2d, agg_out, deg_out,
                sbuf, dbuf, rows, zbuf, deg_local, dred, accum, deg_stage,
                sem_i, sem_p, sg0, sg1, sg2, sg3, ss0, ss1, ss2, ss3):
    sem_g, sem_s = (sg0, sg1, sg2, sg3), (ss0, ss1, ss2, ss3)
    cid = lax.axis_index("c")
    sid = lax.axis_index("s")
    wid = sid * NC + cid
    row0 = sid * RPT
    ch0, nch = _worker_range(wid)

    _load_idx(e2d, sbuf, dbuf, sem_i, ch0, wid)
    _zero_fill(zbuf, RPT)
    pltpu.sync_copy(zbuf, accum.at[pl.ds(row0, RPT)])

    def fz(i, _):
        deg_local[pl.ds(i * H, H)] = jnp.zeros((H,), jnp.float32)
        return 0
    lax.fori_loop(0, N_PAD // H, fz, 0)
    _drain_idx(e2d, sbuf, dbuf, sem_i, ch0, wid)
    plsc.subcore_barrier()

    _agg_pipeline(table, sbuf, dbuf, rows, nch, accum, deg_local,
                  sem_g, sem_s)

    pltpu.sync_copy(deg_local, deg_stage.at[sid])
    plsc.subcore_barrier()
    for k in range(NS):
        pltpu.async_copy(deg_stage.at[k, pl.ds(row0, RPT)], dred.at[k],
                         sem_p)
    for k in range(NS):
        pltpu.make_async_copy(deg_stage.at[k, pl.ds(row0, RPT)],
                              dred.at[k], sem_p).wait()

    def fr(i, _):
        acc = dred[0, pl.ds(i * H, H)]
        for k in range(1, NS):
            acc = acc + dred[k, pl.ds(i * H, H)]
        deg_local[pl.ds(i * H, H)] = acc
        return 0
    lax.fori_loop(0, RPT // H, fr, 0)

    out_off = cid * N_PAD + row0
    pltpu.sync_copy(deg_local.at[pl.ds(0, RPT)],
                    deg_out.at[pl.ds(out_off, RPT)])
    pltpu.sync_copy(accum.at[pl.ds(row0, RPT)], zbuf)
    pltpu.sync_copy(zbuf, agg_out.at[pl.ds(out_off, RPT)])


def _sc_l2_body(agg_in, deg_in, e2d, b1h, mu_out,
                sbuf, dbuf, rows, zbuf, a0, a1, d0, d1, b1v, htab, accum,
                sem_i, sem_p, sg0, sg1, sg2, sg3, ss0, ss1, ss2, ss3):
    sem_g, sem_s = (sg0, sg1, sg2, sg3), (ss0, ss1, ss2, ss3)
    cid = lax.axis_index("c")
    sid = lax.axis_index("s")
    wid = sid * NC + cid
    row0 = sid * RPT
    ch0, nch = _worker_range(wid)

    _load_idx(e2d, sbuf, dbuf, sem_i, ch0, wid)
    pltpu.async_copy(agg_in.at[pl.ds(row0, RPT)], a0, sem_p)
    pltpu.async_copy(agg_in.at[pl.ds(N_PAD + row0, RPT)], a1, sem_p)
    pltpu.async_copy(deg_in.at[pl.ds(row0, RPT)], d0, sem_p)
    pltpu.async_copy(deg_in.at[pl.ds(N_PAD + row0, RPT)], d1, sem_p)
    pltpu.async_copy(b1h, b1v, sem_p)

    _zero_fill(zbuf, RPT)
    pltpu.sync_copy(zbuf, accum.at[pl.ds(row0, RPT)])

    pltpu.make_async_copy(agg_in.at[pl.ds(row0, RPT)], a0, sem_p).wait()
    pltpu.make_async_copy(agg_in.at[pl.ds(row0, RPT)], a1, sem_p).wait()
    pltpu.make_async_copy(deg_in.at[pl.ds(row0, RPT)], d0, sem_p).wait()
    pltpu.make_async_copy(deg_in.at[pl.ds(row0, RPT)], d1, sem_p).wait()
    pltpu.make_async_copy(b1h, b1v, sem_p).wait()

    bvec = b1v[0]

    def hblk(i, _):
        dv = jnp.maximum(d0[pl.ds(i * H, H)] + d1[pl.ds(i * H, H)], 1.0)
        rv = jnp.ones((H,), jnp.float32) / dv
        for m in range(H):
            r = i * H + m
            rm = jnp.full((H,), rv[m], jnp.float32)
            a0[r] = jnp.maximum((a0[r] + a1[r]) * rm + bvec, 0.0)
        return 0
    lax.fori_loop(0, RPT // H, hblk, 0)
    pltpu.sync_copy(a0, htab.at[pl.ds(row0, RPT)])
    _drain_idx(e2d, sbuf, dbuf, sem_i, ch0, wid)
    plsc.subcore_barrier()

    _agg_pipeline(htab, sbuf, dbuf, rows, nch, accum, None, sem_g, sem_s)

    plsc.subcore_barrier()
    pltpu.sync_copy(accum.at[pl.ds(row0, RPT)], zbuf)

    def mblk(i, _):
        dv = jnp.maximum(d0[pl.ds(i * H, H)] + d1[pl.ds(i * H, H)], 1.0)
        rv = jnp.ones((H,), jnp.float32) / dv
        for m in range(H):
            r = i * H + m
            zbuf[r] = zbuf[r] * jnp.full((H,), rv[m], jnp.float32)
        return 0
    lax.fori_loop(0, RPT // H, mblk, 0)
    pltpu.sync_copy(zbuf, mu_out.at[pl.ds(cid * N_PAD + row0, RPT)])


_SC_MESH = plsc.VectorSubcoreMesh(
    core_axis_name="c", subcore_axis_name="s",
    num_cores=NC, num_subcores=NS)
_SC_PARAMS = pltpu.CompilerParams(use_tc_tiling_on_sc=False,
                                  needs_layout_passes=False)


def _make_sc_l1():
    scratch = [
        pltpu.VMEM(((CH_BASE + 1) * KC,), jnp.int32),
        pltpu.VMEM(((CH_BASE + 1) * KC,), jnp.int32),
        pltpu.VMEM((NSLOT, KC, H), jnp.float32),
        pltpu.VMEM((RPT, H), jnp.float32),
        pltpu.VMEM((N_PAD,), jnp.float32),
        pltpu.VMEM((NS, RPT), jnp.float32),
        pltpu.VMEM_SHARED((N_PAD, H), jnp.float32),
        pltpu.VMEM_SHARED((NS, N_PAD), jnp.float32),
    ] + [pltpu.SemaphoreType.DMA] * 10
    return pl.kernel(
        _sc_l1_body,
        out_type=(jax.ShapeDtypeStruct((NC * N_PAD, H), jnp.float32),
                  jax.ShapeDtypeStruct((NC * N_PAD,), jnp.float32)),
        mesh=_SC_MESH,
        scratch_types=scratch,
        compiler_params=_SC_PARAMS,
    )


def _make_sc_l2():
    scratch = [
        pltpu.VMEM(((CH_BASE + 1) * KC,), jnp.int32),
        pltpu.VMEM(((CH_BASE + 1) * KC,), jnp.int32),
        pltpu.VMEM((NSLOT, KC, H), jnp.float32),
        pltpu.VMEM((RPT, H), jnp.float32),
        pltpu.VMEM((RPT, H), jnp.float32),
        pltpu.VMEM((RPT, H), jnp.float32),
        pltpu.VMEM((RPT,), jnp.float32),
        pltpu.VMEM((RPT,), jnp.float32),
        pltpu.VMEM((1, H), jnp.float32),
        pltpu.VMEM_SHARED((N_PAD, H), jnp.float32),
        pltpu.VMEM_SHARED((N_PAD, H), jnp.float32),
    ] + [pltpu.SemaphoreType.DMA] * 10
    return pl.kernel(
        _sc_l2_body,
        out_type=jax.ShapeDtypeStruct((NC * N_PAD, H), jnp.float32),
        mesh=_SC_MESH,
        scratch_types=scratch,
        compiler_params=_SC_PARAMS,
    )


def _mm_body(x_ref, w_ref, o_ref):
    o_ref[...] = jnp.dot(x_ref[...], w_ref[...],
                         preferred_element_type=jnp.float32)


def _out_body(m0, m1, w_ref, b_ref, o_ref):
    z = jnp.dot(m0[...] + m1[...], w_ref[...],
                preferred_element_type=jnp.float32) + b_ref[...]
    m = jnp.max(z, axis=1, keepdims=True)
    lse = jnp.log(jnp.sum(jnp.exp(z - m), axis=1, keepdims=True)) + m
    o_ref[...] = z - lse


def kernel(x, edge_index, W1, b1, W2, b2):
    e2d = edge_index
    NB = N_PAD // 1024

    y1p = pl.pallas_call(
        _mm_body,
        grid=(5,),
        in_specs=[pl.BlockSpec((2048, D), lambda i: (i, 0)),
                  pl.BlockSpec((D, H), lambda i: (0, 0))],
        out_specs=pl.BlockSpec((2048, H), lambda i: (i, 0)),
        out_shape=jax.ShapeDtypeStruct((N_PAD, H), jnp.float32),
    )(x, W1)

    agg1, degp = _make_sc_l1()(y1p, e2d)

    mu2 = _make_sc_l2()(agg1, degp, e2d, b1.reshape(1, H))

    bspec = lambda off: pl.BlockSpec((2048, H), lambda i: (i + off, 0))
    out = pl.pallas_call(
        _out_body,
        grid=(5,),
        in_specs=[bspec(0), bspec(5),
                  pl.BlockSpec((H, C), lambda i: (0, 0)),
                  pl.BlockSpec((1, C), lambda i: (0, 0))],
        out_specs=pl.BlockSpec((2048, C), lambda i: (i, 0)),
        out_shape=jax.ShapeDtypeStruct((N, C), jnp.float32),
    )(mu2, mu2, W2, b2.reshape(1, C))
    return out


# speedup vs baseline: 1.0359x; 1.0022x over previous
import functools

import jax
import jax.numpy as jnp
from jax import lax
from jax.experimental import pallas as pl
from jax.experimental.pallas import tpu as pltpu
from jax.experimental.pallas import tpu_sc as plsc

N = 10000
E = 320000
D = 128
H = 16
C = 40

NC, NS, L = 2, 16, 16
NW = NC * NS
N_PAD = 10240
KC = 512
NCH = E // KC
CH_BASE = NCH // NW
CH_EXTRA = NCH - CH_BASE * NW
RPT = N_PAD // NS
NSLOT = 4
NSTEP = 4 * ((CH_BASE + 1 + 2) // 4 + 1)


def _worker_range(wid):
    nch = CH_BASE + (wid < CH_EXTRA).astype(jnp.int32)
    ch0 = wid * CH_BASE + jnp.minimum(wid, CH_EXTRA)
    return ch0, nch


def _load_idx(e2d, sbuf, dbuf, sem_i, ch0, wid):
    pltpu.async_copy(e2d.at[0, pl.ds(ch0 * KC, CH_BASE * KC)],
                     sbuf.at[pl.ds(0, CH_BASE * KC)], sem_i)
    pltpu.async_copy(e2d.at[1, pl.ds(ch0 * KC, CH_BASE * KC)],
                     dbuf.at[pl.ds(0, CH_BASE * KC)], sem_i)

    @pl.when(wid < CH_EXTRA)
    def _():
        pltpu.async_copy(e2d.at[0, pl.ds((ch0 + CH_BASE) * KC, KC)],
                         sbuf.at[pl.ds(CH_BASE * KC, KC)], sem_i)
        pltpu.async_copy(e2d.at[1, pl.ds((ch0 + CH_BASE) * KC, KC)],
                         dbuf.at[pl.ds(CH_BASE * KC, KC)], sem_i)


def _drain_idx(e2d, sbuf, dbuf, sem_i, ch0, wid):
    pltpu.make_async_copy(e2d.at[0, pl.ds(ch0 * KC, CH_BASE * KC)],
                          sbuf.at[pl.ds(0, CH_BASE * KC)], sem_i).wait()
    pltpu.make_async_copy(e2d.at[1, pl.ds(ch0 * KC, CH_BASE * KC)],
                          dbuf.at[pl.ds(0, CH_BASE * KC)], sem_i).wait()

    @pl.when(wid < CH_EXTRA)
    def _():
        pltpu.make_async_copy(e2d.at[0, pl.ds((ch0 + CH_BASE) * KC, KC)],
                              sbuf.at[pl.ds(CH_BASE * KC, KC)], sem_i).wait()
        pltpu.make_async_copy(e2d.at[1, pl.ds((ch0 + CH_BASE) * KC, KC)],
                              dbuf.at[pl.ds(CH_BASE * KC, KC)], sem_i).wait()


def _agg_pipeline(table, sbuf, dbuf, rows, nch, accum, deg_local,
                  sem_g, sem_s):
    ones16 = jnp.ones((H,), jnp.float32)

    def step(c, q):
        q2 = (q + 2) % NSLOT

        @pl.when(jnp.logical_and(c >= 2, c - 2 < nch))
        def _():
            pltpu.make_async_copy(
                rows.at[q2], accum.at[dbuf.at[pl.ds(0, KC)]],
                sem_s[q2]).wait()

        @pl.when(c + 2 < nch)
        def _():
            pltpu.async_copy(table.at[sbuf.at[pl.ds((c + 2) * KC, KC)]],
                             rows.at[q2], sem_g[q2])

        @pl.when(c < nch)
        def _():
            if deg_local is not None:
                for k in range(KC // H):
                    dvec = dbuf[pl.ds(c * KC + k * H, H)]
                    plsc.addupdate_scatter(deg_local, [dvec], ones16)
            pltpu.make_async_copy(
                table.at[sbuf.at[pl.ds(0, KC)]], rows.at[q],
                sem_g[q]).wait()
            pltpu.async_copy(rows.at[q],
                             accum.at[dbuf.at[pl.ds(c * KC, KC)]],
                             sem_s[q], add=True)

    pltpu.async_copy(table.at[sbuf.at[pl.ds(0, KC)]], rows.at[0], sem_g[0])
    pltpu.async_copy(table.at[sbuf.at[pl.ds(KC, KC)]], rows.at[1], sem_g[1])

    def outer(i, _):
        for q in range(NSLOT):
            step(i * NSLOT + q, q)
        return 0
    lax.fori_loop(0, NSTEP // NSLOT, outer, 0)


def _zero_fill(buf, n):
    def f(i, _):
        buf[i] = jnp.zeros((H,), jnp.float32)
        return 0
    lax.fori_loop(0, n, f, 0)


def _sc_l1_body(table, e2d, agg_out, deg_out,
                sbuf, dbuf, rows, zbuf, deg_local, dred, accum, deg_stage,
                sem_i, sem_p, sg0, sg1, sg2, sg3, ss0, ss1, ss2, ss3):
    sem_g, sem_s = (sg0, sg1, sg2, sg3), (ss0, ss1, ss2, ss3)
    cid = lax.axis_index("c")
    sid = lax.axis_index("s")
    wid = sid * NC + cid
    row0 = sid * RPT
    ch0, nch = _worker_range(wid)

    _load_idx(e2d, sbuf, dbuf, sem_i, ch0, wid)
    _zero_fill(zbuf, RPT)
    pltpu.sync_copy(zbuf, accum.at[pl.ds(row0, RPT)])

    def fz(i, _):
        deg_local[pl.ds(i * H, H)] = jnp.zeros((H,), jnp.float32)
        return 0
    lax.fori_loop(0, N_PAD // H, fz, 0)
    _drain_idx(e2d, sbuf, dbuf, sem_i, ch0, wid)
    plsc.subcore_barrier()

    _agg_pipeline(table, sbuf, dbuf, rows, nch, accum, deg_local,
                  sem_g, sem_s)

    pltpu.sync_copy(deg_local, deg_stage.at[sid])
    plsc.subcore_barrier()
    for k in range(NS):
        pltpu.async_copy(deg_stage.at[k, pl.ds(row0, RPT)], dred.at[k],
                         sem_p)
    for k in range(NS):
        pltpu.make_async_copy(deg_stage.at[k, pl.ds(row0, RPT)],
                              dred.at[k], sem_p).wait()

    def fr(i, _):
        acc = dred[0, pl.ds(i * H, H)]
        for k in range(1, NS):
            acc = acc + dred[k, pl.ds(i * H, H)]
        deg_local[pl.ds(i * H, H)] = acc
        return 0
    lax.fori_loop(0, RPT // H, fr, 0)

    out_off = cid * N_PAD + row0
    pltpu.sync_copy(deg_local.at[pl.ds(0, RPT)],
                    deg_out.at[pl.ds(out_off, RPT)])
    pltpu.sync_copy(accum.at[pl.ds(row0, RPT)],
                    agg_out.at[pl.ds(out_off, RPT)])


def _sc_l2_body(agg_in, deg_in, e2d, b1h, mu_out,
                sbuf, dbuf, rows, zbuf, a0, a1, d0, d1, b1v, htab, accum,
                sem_i, sem_p, sg0, sg1, sg2, sg3, ss0, ss1, ss2, ss3):
    sem_g, sem_s = (sg0, sg1, sg2, sg3), (ss0, ss1, ss2, ss3)
    cid = lax.axis_index("c")
    sid = lax.axis_index("s")
    wid = sid * NC + cid
    row0 = sid * RPT
    ch0, nch = _worker_range(wid)

    _load_idx(e2d, sbuf, dbuf, sem_i, ch0, wid)
    pltpu.async_copy(agg_in.at[pl.ds(row0, RPT)], a0, sem_p)
    pltpu.async_copy(agg_in.at[pl.ds(N_PAD + row0, RPT)], a1, sem_p)
    pltpu.async_copy(deg_in.at[pl.ds(row0, RPT)], d0, sem_p)
    pltpu.async_copy(deg_in.at[pl.ds(N_PAD + row0, RPT)], d1, sem_p)
    pltpu.async_copy(b1h, b1v, sem_p)

    _zero_fill(zbuf, RPT)
    pltpu.sync_copy(zbuf, accum.at[pl.ds(row0, RPT)])

    pltpu.make_async_copy(agg_in.at[pl.ds(row0, RPT)], a0, sem_p).wait()
    pltpu.make_async_copy(agg_in.at[pl.ds(row0, RPT)], a1, sem_p).wait()
    pltpu.make_async_copy(deg_in.at[pl.ds(row0, RPT)], d0, sem_p).wait()
    pltpu.make_async_copy(deg_in.at[pl.ds(row0, RPT)], d1, sem_p).wait()
    pltpu.make_async_copy(b1h, b1v, sem_p).wait()

    bvec = b1v[0]

    def hblk(i, _):
        dv = jnp.maximum(d0[pl.ds(i * H, H)] + d1[pl.ds(i * H, H)], 1.0)
        rv = jnp.ones((H,), jnp.float32) / dv
        for m in range(H):
            r = i * H + m
            rm = jnp.full((H,), rv[m], jnp.float32)
            a0[r] = jnp.maximum((a0[r] + a1[r]) * rm + bvec, 0.0)
        return 0
    lax.fori_loop(0, RPT // H, hblk, 0)
    pltpu.sync_copy(a0, htab.at[pl.ds(row0, RPT)])
    _drain_idx(e2d, sbuf, dbuf, sem_i, ch0, wid)
    plsc.subcore_barrier()

    _agg_pipeline(htab, sbuf, dbuf, rows, nch, accum, None, sem_g, sem_s)

    plsc.subcore_barrier()
    pltpu.sync_copy(accum.at[pl.ds(row0, RPT)], zbuf)

    def mblk(i, _):
        dv = jnp.maximum(d0[pl.ds(i * H, H)] + d1[pl.ds(i * H, H)], 1.0)
        rv = jnp.ones((H,), jnp.float32) / dv
        for m in range(H):
            r = i * H + m
            zbuf[r] = zbuf[r] * jnp.full((H,), rv[m], jnp.float32)
        return 0
    lax.fori_loop(0, RPT // H, mblk, 0)
    pltpu.sync_copy(zbuf, mu_out.at[pl.ds(cid * N_PAD + row0, RPT)])


_SC_MESH = plsc.VectorSubcoreMesh(
    core_axis_name="c", subcore_axis_name="s",
    num_cores=NC, num_subcores=NS)
_SC_PARAMS = pltpu.CompilerParams(use_tc_tiling_on_sc=False,
                                  needs_layout_passes=False)


def _make_sc_l1():
    scratch = [
        pltpu.VMEM(((CH_BASE + 1) * KC,), jnp.int32),
        pltpu.VMEM(((CH_BASE + 1) * KC,), jnp.int32),
        pltpu.VMEM((NSLOT, KC, H), jnp.float32),
        pltpu.VMEM((RPT, H), jnp.float32),
        pltpu.VMEM((N_PAD,), jnp.float32),
        pltpu.VMEM((NS, RPT), jnp.float32),
        pltpu.VMEM_SHARED((N_PAD, H), jnp.float32),
        pltpu.VMEM_SHARED((NS, N_PAD), jnp.float32),
    ] + [pltpu.SemaphoreType.DMA] * 10
    return pl.kernel(
        _sc_l1_body,
        out_type=(jax.ShapeDtypeStruct((NC * N_PAD, H), jnp.float32),
                  jax.ShapeDtypeStruct((NC * N_PAD,), jnp.float32)),
        mesh=_SC_MESH,
        scratch_types=scratch,
        compiler_params=_SC_PARAMS,
    )


def _make_sc_l2():
    scratch = [
        pltpu.VMEM(((CH_BASE + 1) * KC,), jnp.int32),
        pltpu.VMEM(((CH_BASE + 1) * KC,), jnp.int32),
        pltpu.VMEM((NSLOT, KC, H), jnp.float32),
        pltpu.VMEM((RPT, H), jnp.float32),
        pltpu.VMEM((RPT, H), jnp.float32),
        pltpu.VMEM((RPT, H), jnp.float32),
        pltpu.VMEM((RPT,), jnp.float32),
        pltpu.VMEM((RPT,), jnp.float32),
        pltpu.VMEM((1, H), jnp.float32),
        pltpu.VMEM_SHARED((N_PAD, H), jnp.float32),
        pltpu.VMEM_SHARED((N_PAD, H), jnp.float32),
    ] + [pltpu.SemaphoreType.DMA] * 10
    return pl.kernel(
        _sc_l2_body,
        out_type=jax.ShapeDtypeStruct((NC * N_PAD, H), jnp.float32),
        mesh=_SC_MESH,
        scratch_types=scratch,
        compiler_params=_SC_PARAMS,
    )


def _mm_body(x_ref, w_ref, o_ref):
    o_ref[...] = jnp.dot(x_ref[...], w_ref[...],
                         preferred_element_type=jnp.float32)


def _out_body(m0, m1, w_ref, b_ref, o_ref):
    z = jnp.dot(m0[...] + m1[...], w_ref[...],
                preferred_element_type=jnp.float32) + b_ref[...]
    m = jnp.max(z, axis=1, keepdims=True)
    lse = jnp.log(jnp.sum(jnp.exp(z - m), axis=1, keepdims=True)) + m
    o_ref[...] = z - lse


def kernel(x, edge_index, W1, b1, W2, b2):
    e2d = edge_index
    NB = N_PAD // 1024

    y1p = pl.pallas_call(
        _mm_body,
        grid=(5,),
        in_specs=[pl.BlockSpec((2048, D), lambda i: (i, 0)),
                  pl.BlockSpec((D, H), lambda i: (0, 0))],
        out_specs=pl.BlockSpec((2048, H), lambda i: (i, 0)),
        out_shape=jax.ShapeDtypeStruct((N_PAD, H), jnp.float32),
    )(x, W1)

    agg1, degp = _make_sc_l1()(y1p, e2d)

    mu2 = _make_sc_l2()(agg1, degp, e2d, b1.reshape(1, H))

    bspec = lambda off: pl.BlockSpec((2048, H), lambda i: (i + off, 0))
    out = pl.pallas_call(
        _out_body,
        grid=(5,),
        in_specs=[bspec(0), bspec(5),
                  pl.BlockSpec((H, C), lambda i: (0, 0)),
                  pl.BlockSpec((1, C), lambda i: (0, 0))],
        out_specs=pl.BlockSpec((2048, C), lambda i: (i, 0)),
        out_shape=jax.ShapeDtypeStruct((N, C), jnp.float32),
    )(mu2, mu2, W2, b2.reshape(1, C))
    return out
